# baseline (speedup 1.0000x reference)
import functools

import jax
import jax.numpy as jnp
from jax import lax
from jax.experimental import pallas as pl
from jax.experimental.pallas import tpu as pltpu
from jax.experimental.pallas import tpu_sc as plsc

N = 10000
E = 320000
D = 128

NC = 2
NS = 16
NW = NC * NS
WIN = 128
WT = E // WIN
WPT = WT // NW
EXTRA = WT - WPT * NW
NBUF = 3

DCH = 632
DCH_LAST = N - (NS - 1) * DCH
TAIL = DCH - 4 * WIN
TAIL_LAST = DCH_LAST - 4 * WIN

_INTERPRET = False


def _mesh():
  return plsc.VectorSubcoreMesh(
      core_axis_name="c", subcore_axis_name="s",
      num_cores=NC, num_subcores=NS)


def _worker(cid, tid):
  wid = cid * NS + tid
  ebase = (wid * WPT + jnp.minimum(wid, EXTRA)) * WIN
  has_extra = wid < EXTRA
  return ebase, has_extra


def _deg_kernel(dst_hbm, out_hbm, didx, ones_v, zbuf, acc, isem, ssem):
  cid = lax.axis_index("c")
  tid = lax.axis_index("s")
  ebase, has_extra = _worker(cid, tid)

  def load(w, carry):
    pltpu.async_copy(dst_hbm.at[pl.ds(ebase + w * WIN, WIN)],
                     didx.at[w], isem)
    return carry
  lax.fori_loop(0, WPT, load, 0)

  @pl.when(has_extra)
  def _():
    pltpu.async_copy(dst_hbm.at[pl.ds(ebase + WPT * WIN, WIN)],
                     didx.at[WPT], isem)

  for j in range(WIN // 16):
    ones_v[pl.ds(j * 16, 16)] = jnp.full((16,), 1.0, jnp.float32)
    zbuf[pl.ds(j * 16, 16)] = jnp.zeros((16,), jnp.float32)

  cbase = tid * DCH
  for j in range(4):
    pltpu.sync_copy(zbuf, acc.at[pl.ds(cbase + j * WIN, WIN)])

  @pl.when(tid < NS - 1)
  def _():
    pltpu.sync_copy(zbuf.at[pl.ds(0, TAIL)],
                    acc.at[pl.ds(cbase + 4 * WIN, TAIL)])

  @pl.when(tid == NS - 1)
  def _():
    pltpu.sync_copy(zbuf.at[pl.ds(0, TAIL_LAST)],
                    acc.at[pl.ds(cbase + 4 * WIN, TAIL_LAST)])

  def idrain(w, carry):
    pltpu.make_async_copy(dst_hbm.at[pl.ds(0, WIN)], didx.at[0],
                          isem).wait()
    return carry
  lax.fori_loop(0, WPT, idrain, 0)

  @pl.when(has_extra)
  def _():
    pltpu.make_async_copy(dst_hbm.at[pl.ds(0, WIN)], didx.at[0],
                          isem).wait()

  plsc.subcore_barrier()

  def fire(w, carry):
    pltpu.async_copy(ones_v, acc.at[didx.at[w]], ssem, add=True)
    return carry
  lax.fori_loop(0, WPT, fire, 0)

  @pl.when(has_extra)
  def _():
    pltpu.async_copy(ones_v, acc.at[didx.at[WPT]], ssem, add=True)

  def sdrain(w, carry):
    pltpu.make_async_copy(dst_hbm.at[pl.ds(0, WIN)], didx.at[0],
                          ssem).wait()
    return carry
  lax.fori_loop(0, WPT, sdrain, 0)

  @pl.when(has_extra)
  def _():
    pltpu.make_async_copy(dst_hbm.at[pl.ds(0, WIN)], didx.at[0],
                          ssem).wait()

  plsc.subcore_barrier()

  obase = cid * N + tid * DCH
  for j in range(4):
    pltpu.sync_copy(acc.at[pl.ds(cbase + j * WIN, WIN)], zbuf)
    pltpu.sync_copy(zbuf, out_hbm.at[pl.ds(obase + j * WIN, WIN)])

  @pl.when(tid < NS - 1)
  def _():
    pltpu.sync_copy(acc.at[pl.ds(cbase + 4 * WIN, TAIL)],
                    zbuf.at[pl.ds(0, TAIL)])
    pltpu.sync_copy(zbuf.at[pl.ds(0, TAIL)],
                    out_hbm.at[pl.ds(obase + 4 * WIN, TAIL)])

  @pl.when(tid == NS - 1)
  def _():
    pltpu.sync_copy(acc.at[pl.ds(cbase + 4 * WIN, TAIL_LAST)],
                    zbuf.at[pl.ds(0, TAIL_LAST)])
    pltpu.sync_copy(zbuf.at[pl.ds(0, TAIL_LAST)],
                    out_hbm.at[pl.ds(obase + 4 * WIN, TAIL_LAST)])


@functools.cache
def _deg_call():
  return pl.kernel(
      _deg_kernel,
      out_type=jax.ShapeDtypeStruct((NC * N,), jnp.float32),
      mesh=_mesh(),
      scratch_types=[
          pltpu.VMEM((WPT + 1, WIN), jnp.int32),
          pltpu.VMEM((WIN,), jnp.float32),
          pltpu.VMEM((WIN,), jnp.float32),
          pltpu.VMEM_SHARED((N,), jnp.float32),
          pltpu.SemaphoreType.DMA,
          pltpu.SemaphoreType.DMA,
      ],
      interpret=_INTERPRET,
  )


def _agg_kernel(y_hbm, src_hbm, dst_hbm, zeros_hbm, out_hbm,
                sidx, didx, rows0, rows1, rows2, acc,
                isem0, isem1, isem2, gsem0, gsem1, gsem2,
                ssem0, ssem1, ssem2):
  cid = lax.axis_index("c")
  tid = lax.axis_index("s")
  ebase, has_extra = _worker(cid, tid)
  nwt = WPT + has_extra.astype(jnp.int32)

  rows = (rows0, rows1, rows2)
  isem = (isem0, isem1, isem2)
  gsem = (gsem0, gsem1, gsem2)
  ssem = (ssem0, ssem1, ssem2)

  def start_idx(b, w):
    pltpu.async_copy(src_hbm.at[pl.ds(ebase + w * WIN, WIN)],
                     sidx.at[b], isem[b])
    pltpu.async_copy(dst_hbm.at[pl.ds(ebase + w * WIN, WIN)],
                     didx.at[b], isem[b])

  def wait_idx(b):
    pltpu.make_async_copy(src_hbm.at[pl.ds(0, WIN)], sidx.at[0],
                          isem[b]).wait()
    pltpu.make_async_copy(src_hbm.at[pl.ds(0, WIN)], sidx.at[0],
                          isem[b]).wait()

  HW = WIN // 2

  def start_gather(b):
    pltpu.async_copy(y_hbm.at[sidx.at[b].at[pl.ds(0, HW)]],
                     rows[b].at[pl.ds(0, HW)], gsem[b])
    pltpu.async_copy(y_hbm.at[sidx.at[b].at[pl.ds(HW, HW)]],
                     rows[b].at[pl.ds(HW, HW)], gsem[b])

  def wait_gather(b):
    pltpu.make_async_copy(y_hbm.at[pl.ds(0, WIN)], rows[b],
                          gsem[b]).wait()

  def start_scatter(b):
    pltpu.async_copy(rows[b], acc.at[didx.at[b]], ssem[b], add=True)

  def wait_scatter(b):
    pltpu.make_async_copy(y_hbm.at[pl.ds(0, WIN)], rows[b],
                          ssem[b]).wait()

  start_idx(0, 0)
  start_idx(1, 1)

  cbase = tid * DCH
  pltpu.sync_copy(zeros_hbm, rows0)
  for j in range(4):
    pltpu.sync_copy(rows0, acc.at[pl.ds(cbase + j * WIN, WIN)])

  @pl.when(tid < NS - 1)
  def _():
    pltpu.sync_copy(rows0.at[pl.ds(0, TAIL)],
                    acc.at[pl.ds(cbase + 4 * WIN, TAIL)])

  @pl.when(tid == NS - 1)
  def _():
    pltpu.sync_copy(rows0.at[pl.ds(0, TAIL_LAST)],
                    acc.at[pl.ds(cbase + 4 * WIN, TAIL_LAST)])

  plsc.subcore_barrier()

  wait_idx(0)
  start_gather(0)

  def group(g, carry):
    for b in range(NBUF):
      w = NBUF * g + b
      b1 = (b + 1) % NBUF
      b2 = (b + 2) % NBUF
      w1 = w + 1
      w2 = w + 2

      @pl.when(w2 < nwt)
      def _():
        @pl.when(w2 >= NBUF)
        def _():
          wait_scatter(b2)
        start_idx(b2, w2)

      @pl.when(w1 < nwt)
      def _():
        wait_idx(b1)
        start_gather(b1)

      wait_gather(b)
      start_scatter(b)
    return carry

  lax.fori_loop(0, WPT // NBUF, group, 0)

  @pl.when(has_extra)
  def _():
    wait_gather(0)
    start_scatter(0)

  for b in range(NBUF):
    wait_scatter(b)

  plsc.subcore_barrier()

  for j in range(4):
    pltpu.sync_copy(acc.at[pl.ds(cbase + j * WIN, WIN)], rows0)
    pltpu.sync_copy(rows0, out_hbm.at[cid, pl.ds(cbase + j * WIN, WIN)])

  @pl.when(tid < NS - 1)
  def _():
    pltpu.sync_copy(acc.at[pl.ds(cbase + 4 * WIN, TAIL)],
                    rows0.at[pl.ds(0, TAIL)])
    pltpu.sync_copy(rows0.at[pl.ds(0, TAIL)],
                    out_hbm.at[cid, pl.ds(cbase + 4 * WIN, TAIL)])

  @pl.when(tid == NS - 1)
  def _():
    pltpu.sync_copy(acc.at[pl.ds(cbase + 4 * WIN, TAIL_LAST)],
                    rows0.at[pl.ds(0, TAIL_LAST)])
    pltpu.sync_copy(rows0.at[pl.ds(0, TAIL_LAST)],
                    out_hbm.at[cid, pl.ds(cbase + 4 * WIN, TAIL_LAST)])


@functools.cache
def _agg_call():
  return pl.kernel(
      _agg_kernel,
      out_type=jax.ShapeDtypeStruct((NC, N, D), jnp.float32),
      mesh=_mesh(),
      scratch_types=[
          pltpu.VMEM((NBUF, WIN), jnp.int32),
          pltpu.VMEM((NBUF, WIN), jnp.int32),
          pltpu.VMEM((WIN, D), jnp.float32),
          pltpu.VMEM((WIN, D), jnp.float32),
          pltpu.VMEM((WIN, D), jnp.float32),
          pltpu.VMEM_SHARED((N, D), jnp.float32),
          pltpu.SemaphoreType.DMA,
          pltpu.SemaphoreType.DMA,
          pltpu.SemaphoreType.DMA,
          pltpu.SemaphoreType.DMA,
          pltpu.SemaphoreType.DMA,
          pltpu.SemaphoreType.DMA,
          pltpu.SemaphoreType.DMA,
          pltpu.SemaphoreType.DMA,
          pltpu.SemaphoreType.DMA,
      ],
      interpret=_INTERPRET,
  )


RB = 400


def _dot(a, b):
  return lax.dot_general(a, b, (((1,), (0,)), ((), ())),
                         precision=lax.Precision.HIGHEST,
                         preferred_element_type=jnp.float32)


def _k2_body(degp_ref, x_ref, w_ref, dinv_ref, y1_ref):
  d = degp_ref[0] + degp_ref[1] + 1.0
  dinv = lax.rsqrt(jnp.maximum(d, 1.0))
  dinv_ref[...] = dinv
  y1_ref[...] = _dot(x_ref[...], w_ref[...]) * dinv


def _k2(degp, x, w1):
  return pl.pallas_call(
      _k2_body,
      grid=(N // RB,),
      in_specs=[
          pl.BlockSpec((NC, RB, 1), lambda i: (0, i, 0)),
          pl.BlockSpec((RB, D), lambda i: (i, 0)),
          pl.BlockSpec((D, D), lambda i: (0, 0)),
      ],
      out_specs=[
          pl.BlockSpec((RB, 1), lambda i: (i, 0)),
          pl.BlockSpec((RB, D), lambda i: (i, 0)),
      ],
      out_shape=[
          jax.ShapeDtypeStruct((N, 1), jnp.float32),
          jax.ShapeDtypeStruct((N, D), jnp.float32),
      ],
      interpret=_INTERPRET,
  )(degp, x, w1)


def _k4_body(s_ref, y1_ref, dinv_ref, w_ref, b_ref, y2_ref):
  dinv = dinv_ref[...]
  h1 = dinv * (s_ref[0] + s_ref[1] + y1_ref[...]) + b_ref[...]
  y2_ref[...] = _dot(h1, w_ref[...]) * dinv


def _k4(s1, y1, dinv, w2, b1):
  return pl.pallas_call(
      _k4_body,
      grid=(N // RB,),
      in_specs=[
          pl.BlockSpec((NC, RB, D), lambda i: (0, i, 0)),
          pl.BlockSpec((RB, D), lambda i: (i, 0)),
          pl.BlockSpec((RB, 1), lambda i: (i, 0)),
          pl.BlockSpec((D, D), lambda i: (0, 0)),
          pl.BlockSpec((1, D), lambda i: (0, 0)),
      ],
      out_specs=pl.BlockSpec((RB, D), lambda i: (i, 0)),
      out_shape=jax.ShapeDtypeStruct((N, D), jnp.float32),
      interpret=_INTERPRET,
  )(s1, y1, dinv, w2, b1)


def _k6_body(s_ref, y2_ref, dinv_ref, b_ref, out_ref):
  out_ref[...] = (dinv_ref[...] * (s_ref[0] + s_ref[1] + y2_ref[...])
                  + b_ref[...])


def _k6(s2, y2, dinv, b2):
  return pl.pallas_call(
      _k6_body,
      grid=(N // RB,),
      in_specs=[
          pl.BlockSpec((NC, RB, D), lambda i: (0, i, 0)),
          pl.BlockSpec((RB, D), lambda i: (i, 0)),
          pl.BlockSpec((RB, 1), lambda i: (i, 0)),
          pl.BlockSpec((1, D), lambda i: (0, 0)),
      ],
      out_specs=pl.BlockSpec((RB, D), lambda i: (i, 0)),
      out_shape=jax.ShapeDtypeStruct((N, D), jnp.float32),
      interpret=_INTERPRET,
  )(s2, y2, dinv, b2)


def kernel(node_feature, edge_index, W1, b1, W2, b2):
  src = edge_index[0]
  dst = edge_index[1]
  zeros2d = jnp.zeros((WIN, D), jnp.float32)

  degp = _deg_call()(dst)
  dinv, y1 = _k2(degp.reshape(NC, N, 1), node_feature, W1)
  s1 = _agg_call()(y1, src, dst, zeros2d)
  y2 = _k4(s1, y1, dinv, W2, b1.reshape(1, D))
  s2 = _agg_call()(y2, src, dst, zeros2d)
  return _k6(s2, y2, dinv, b2.reshape(1, D))


# speedup vs baseline: 1.0245x; 1.0245x over previous
import functools

import jax
import jax.numpy as jnp
from jax import lax
from jax.experimental import pallas as pl
from jax.experimental.pallas import tpu as pltpu
from jax.experimental.pallas import tpu_sc as plsc

N = 10000
E = 320000
D = 128

NC = 2
NS = 16
NW = NC * NS
WIN = 128
WT = E // WIN
WPT = WT // NW
EXTRA = WT - WPT * NW
NBUF = 3

DCH = 632
DCH_LAST = N - (NS - 1) * DCH
TAIL = DCH - 4 * WIN
TAIL_LAST = DCH_LAST - 4 * WIN

_INTERPRET = False


def _mesh():
  return plsc.VectorSubcoreMesh(
      core_axis_name="c", subcore_axis_name="s",
      num_cores=NC, num_subcores=NS)


def _worker(cid, tid):
  wid = cid * NS + tid
  ebase = (wid * WPT + jnp.minimum(wid, EXTRA)) * WIN
  has_extra = wid < EXTRA
  return ebase, has_extra


def _deg_kernel(dst_hbm, out_hbm, didx, ones_v, zbuf, acc, isem, ssem):
  cid = lax.axis_index("c")
  tid = lax.axis_index("s")
  ebase, has_extra = _worker(cid, tid)

  def load(w, carry):
    pltpu.async_copy(dst_hbm.at[pl.ds(ebase + w * WIN, WIN)],
                     didx.at[w], isem)
    return carry
  lax.fori_loop(0, WPT, load, 0)

  @pl.when(has_extra)
  def _():
    pltpu.async_copy(dst_hbm.at[pl.ds(ebase + WPT * WIN, WIN)],
                     didx.at[WPT], isem)

  for j in range(WIN // 16):
    ones_v[pl.ds(j * 16, 16)] = jnp.full((16,), 1.0, jnp.float32)
    zbuf[pl.ds(j * 16, 16)] = jnp.zeros((16,), jnp.float32)

  cbase = tid * DCH
  for j in range(4):
    pltpu.sync_copy(zbuf, acc.at[pl.ds(cbase + j * WIN, WIN)])

  @pl.when(tid < NS - 1)
  def _():
    pltpu.sync_copy(zbuf.at[pl.ds(0, TAIL)],
                    acc.at[pl.ds(cbase + 4 * WIN, TAIL)])

  @pl.when(tid == NS - 1)
  def _():
    pltpu.sync_copy(zbuf.at[pl.ds(0, TAIL_LAST)],
                    acc.at[pl.ds(cbase + 4 * WIN, TAIL_LAST)])

  def idrain(w, carry):
    pltpu.make_async_copy(dst_hbm.at[pl.ds(0, WIN)], didx.at[0],
                          isem).wait()
    return carry
  lax.fori_loop(0, WPT, idrain, 0)

  @pl.when(has_extra)
  def _():
    pltpu.make_async_copy(dst_hbm.at[pl.ds(0, WIN)], didx.at[0],
                          isem).wait()

  plsc.subcore_barrier()

  def fire(w, carry):
    pltpu.async_copy(ones_v, acc.at[didx.at[w]], ssem, add=True)
    return carry
  lax.fori_loop(0, WPT, fire, 0)

  @pl.when(has_extra)
  def _():
    pltpu.async_copy(ones_v, acc.at[didx.at[WPT]], ssem, add=True)

  def sdrain(w, carry):
    pltpu.make_async_copy(dst_hbm.at[pl.ds(0, WIN)], didx.at[0],
                          ssem).wait()
    return carry
  lax.fori_loop(0, WPT, sdrain, 0)

  @pl.when(has_extra)
  def _():
    pltpu.make_async_copy(dst_hbm.at[pl.ds(0, WIN)], didx.at[0],
                          ssem).wait()

  plsc.subcore_barrier()

  obase = cid * N + tid * DCH
  for j in range(4):
    pltpu.sync_copy(acc.at[pl.ds(cbase + j * WIN, WIN)], zbuf)
    pltpu.sync_copy(zbuf, out_hbm.at[pl.ds(obase + j * WIN, WIN)])

  @pl.when(tid < NS - 1)
  def _():
    pltpu.sync_copy(acc.at[pl.ds(cbase + 4 * WIN, TAIL)],
                    zbuf.at[pl.ds(0, TAIL)])
    pltpu.sync_copy(zbuf.at[pl.ds(0, TAIL)],
                    out_hbm.at[pl.ds(obase + 4 * WIN, TAIL)])

  @pl.when(tid == NS - 1)
  def _():
    pltpu.sync_copy(acc.at[pl.ds(cbase + 4 * WIN, TAIL_LAST)],
                    zbuf.at[pl.ds(0, TAIL_LAST)])
    pltpu.sync_copy(zbuf.at[pl.ds(0, TAIL_LAST)],
                    out_hbm.at[pl.ds(obase + 4 * WIN, TAIL_LAST)])


@functools.cache
def _deg_call():
  return pl.kernel(
      _deg_kernel,
      out_type=jax.ShapeDtypeStruct((NC * N,), jnp.float32),
      mesh=_mesh(),
      scratch_types=[
          pltpu.VMEM((WPT + 1, WIN), jnp.int32),
          pltpu.VMEM((WIN,), jnp.float32),
          pltpu.VMEM((WIN,), jnp.float32),
          pltpu.VMEM_SHARED((N,), jnp.float32),
          pltpu.SemaphoreType.DMA,
          pltpu.SemaphoreType.DMA,
      ],
      interpret=_INTERPRET,
  )


def _agg_kernel(y_hbm, src_hbm, dst_hbm, zeros_hbm, out_hbm,
                sidx, didx, rows0, rows1, rows2, acc,
                isem0, isem1, isem2, gsem0, gsem1, gsem2,
                ssem0, ssem1, ssem2):
  cid = lax.axis_index("c")
  tid = lax.axis_index("s")
  ebase, has_extra = _worker(cid, tid)
  nwt = WPT + has_extra.astype(jnp.int32)

  rows = (rows0, rows1, rows2)
  isem = (isem0, isem1, isem2)
  gsem = (gsem0, gsem1, gsem2)
  ssem = (ssem0, ssem1, ssem2)

  def start_idx(b, w):
    pltpu.async_copy(src_hbm.at[pl.ds(ebase + w * WIN, WIN)],
                     sidx.at[b], isem[b])
    pltpu.async_copy(dst_hbm.at[pl.ds(ebase + w * WIN, WIN)],
                     didx.at[b], isem[b])

  def wait_idx(b):
    pltpu.make_async_copy(src_hbm.at[pl.ds(0, WIN)], sidx.at[0],
                          isem[b]).wait()
    pltpu.make_async_copy(src_hbm.at[pl.ds(0, WIN)], sidx.at[0],
                          isem[b]).wait()

  HW = WIN // 2

  def start_gather(b):
    pltpu.async_copy(y_hbm.at[sidx.at[b].at[pl.ds(0, HW)]],
                     rows[b].at[pl.ds(0, HW)], gsem[b])
    pltpu.async_copy(y_hbm.at[sidx.at[b].at[pl.ds(HW, HW)]],
                     rows[b].at[pl.ds(HW, HW)], gsem[b])

  def wait_gather(b):
    pltpu.make_async_copy(y_hbm.at[pl.ds(0, WIN)], rows[b],
                          gsem[b]).wait()

  def start_scatter(b):
    pltpu.async_copy(rows[b], acc.at[didx.at[b]], ssem[b], add=True)

  def wait_scatter(b):
    pltpu.make_async_copy(y_hbm.at[pl.ds(0, WIN)], rows[b],
                          ssem[b]).wait()

  start_idx(0, 0)
  start_idx(1, 1)

  cbase = tid * DCH
  pltpu.sync_copy(zeros_hbm, rows0)
  for j in range(4):
    pltpu.async_copy(rows0, acc.at[pl.ds(cbase + j * WIN, WIN)], ssem0)

  @pl.when(tid < NS - 1)
  def _():
    pltpu.async_copy(rows0.at[pl.ds(0, TAIL)],
                     acc.at[pl.ds(cbase + 4 * WIN, TAIL)], ssem0)

  @pl.when(tid == NS - 1)
  def _():
    pltpu.async_copy(rows0.at[pl.ds(0, TAIL_LAST)],
                     acc.at[pl.ds(cbase + 4 * WIN, TAIL_LAST)], ssem0)

  for j in range(4):
    pltpu.make_async_copy(y_hbm.at[pl.ds(0, WIN)], rows0, ssem0).wait()

  @pl.when(tid < NS - 1)
  def _():
    pltpu.make_async_copy(y_hbm.at[pl.ds(0, TAIL)],
                          rows0.at[pl.ds(0, TAIL)], ssem0).wait()

  @pl.when(tid == NS - 1)
  def _():
    pltpu.make_async_copy(y_hbm.at[pl.ds(0, TAIL_LAST)],
                          rows0.at[pl.ds(0, TAIL_LAST)], ssem0).wait()

  plsc.subcore_barrier()

  wait_idx(0)
  start_gather(0)

  def group(g, carry):
    for b in range(NBUF):
      w = NBUF * g + b
      b1 = (b + 1) % NBUF
      b2 = (b + 2) % NBUF
      w1 = w + 1
      w2 = w + 2

      @pl.when(w2 < nwt)
      def _():
        @pl.when(w2 >= NBUF)
        def _():
          wait_scatter(b2)
        start_idx(b2, w2)

      @pl.when(w1 < nwt)
      def _():
        wait_idx(b1)
        start_gather(b1)

      wait_gather(b)
      start_scatter(b)
    return carry

  lax.fori_loop(0, WPT // NBUF, group, 0)

  @pl.when(has_extra)
  def _():
    wait_gather(0)
    start_scatter(0)

  for b in range(NBUF):
    wait_scatter(b)

  plsc.subcore_barrier()

  def wload(b, k):
    pltpu.async_copy(acc.at[pl.ds(cbase + k * WIN, WIN)], rows[b],
                     gsem[b])

  def wload_wait(b):
    pltpu.make_async_copy(y_hbm.at[pl.ds(0, WIN)], rows[b],
                          gsem[b]).wait()

  def wstore(b, k):
    pltpu.async_copy(rows[b], out_hbm.at[cid, pl.ds(cbase + k * WIN, WIN)],
                     ssem[b])

  def wstore_wait(b):
    pltpu.make_async_copy(y_hbm.at[pl.ds(0, WIN)], rows[b],
                          ssem[b]).wait()

  for b in range(3):
    wload(b, b)
  for b in range(3):
    wload_wait(b)
    wstore(b, b)
  wstore_wait(0)
  wload(0, 3)
  wload_wait(0)
  wstore(0, 3)
  wstore_wait(1)

  @pl.when(tid < NS - 1)
  def _():
    pltpu.async_copy(acc.at[pl.ds(cbase + 4 * WIN, TAIL)],
                     rows1.at[pl.ds(0, TAIL)], gsem1)
    pltpu.make_async_copy(y_hbm.at[pl.ds(0, TAIL)],
                          rows1.at[pl.ds(0, TAIL)], gsem1).wait()
    pltpu.async_copy(rows1.at[pl.ds(0, TAIL)],
                     out_hbm.at[cid, pl.ds(cbase + 4 * WIN, TAIL)], ssem1)
    pltpu.make_async_copy(y_hbm.at[pl.ds(0, TAIL)],
                          rows1.at[pl.ds(0, TAIL)], ssem1).wait()

  @pl.when(tid == NS - 1)
  def _():
    pltpu.async_copy(acc.at[pl.ds(cbase + 4 * WIN, TAIL_LAST)],
                     rows1.at[pl.ds(0, TAIL_LAST)], gsem1)
    pltpu.make_async_copy(y_hbm.at[pl.ds(0, TAIL_LAST)],
                          rows1.at[pl.ds(0, TAIL_LAST)], gsem1).wait()
    pltpu.async_copy(rows1.at[pl.ds(0, TAIL_LAST)],
                     out_hbm.at[cid, pl.ds(cbase + 4 * WIN, TAIL_LAST)],
                     ssem1)
    pltpu.make_async_copy(y_hbm.at[pl.ds(0, TAIL_LAST)],
                          rows1.at[pl.ds(0, TAIL_LAST)], ssem1).wait()

  wstore_wait(2)
  wstore_wait(0)


@functools.cache
def _agg_call():
  return pl.kernel(
      _agg_kernel,
      out_type=jax.ShapeDtypeStruct((NC, N, D), jnp.float32),
      mesh=_mesh(),
      scratch_types=[
          pltpu.VMEM((NBUF, WIN), jnp.int32),
          pltpu.VMEM((NBUF, WIN), jnp.int32),
          pltpu.VMEM((WIN, D), jnp.float32),
          pltpu.VMEM((WIN, D), jnp.float32),
          pltpu.VMEM((WIN, D), jnp.float32),
          pltpu.VMEM_SHARED((N, D), jnp.float32),
          pltpu.SemaphoreType.DMA,
          pltpu.SemaphoreType.DMA,
          pltpu.SemaphoreType.DMA,
          pltpu.SemaphoreType.DMA,
          pltpu.SemaphoreType.DMA,
          pltpu.SemaphoreType.DMA,
          pltpu.SemaphoreType.DMA,
          pltpu.SemaphoreType.DMA,
          pltpu.SemaphoreType.DMA,
      ],
      interpret=_INTERPRET,
  )


RB = 400


def _dot(a, b):
  return lax.dot_general(a, b, (((1,), (0,)), ((), ())),
                         preferred_element_type=jnp.float32)


def _k2_body(degp_ref, x_ref, w_ref, dinv_ref, y1_ref):
  d = degp_ref[0] + degp_ref[1] + 1.0
  dinv = lax.rsqrt(jnp.maximum(d, 1.0))
  dinv_ref[...] = dinv
  y1_ref[...] = _dot(x_ref[...], w_ref[...]) * dinv


def _k2(degp, x, w1):
  return pl.pallas_call(
      _k2_body,
      grid=(N // RB,),
      in_specs=[
          pl.BlockSpec((NC, RB, 1), lambda i: (0, i, 0)),
          pl.BlockSpec((RB, D), lambda i: (i, 0)),
          pl.BlockSpec((D, D), lambda i: (0, 0)),
      ],
      out_specs=[
          pl.BlockSpec((RB, 1), lambda i: (i, 0)),
          pl.BlockSpec((RB, D), lambda i: (i, 0)),
      ],
      out_shape=[
          jax.ShapeDtypeStruct((N, 1), jnp.float32),
          jax.ShapeDtypeStruct((N, D), jnp.float32),
      ],
      interpret=_INTERPRET,
  )(degp, x, w1)


def _k4_body(s_ref, y1_ref, dinv_ref, w_ref, b_ref, y2_ref):
  dinv = dinv_ref[...]
  h1 = dinv * (s_ref[0] + s_ref[1] + y1_ref[...]) + b_ref[...]
  y2_ref[...] = _dot(h1, w_ref[...]) * dinv


def _k4(s1, y1, dinv, w2, b1):
  return pl.pallas_call(
      _k4_body,
      grid=(N // RB,),
      in_specs=[
          pl.BlockSpec((NC, RB, D), lambda i: (0, i, 0)),
          pl.BlockSpec((RB, D), lambda i: (i, 0)),
          pl.BlockSpec((RB, 1), lambda i: (i, 0)),
          pl.BlockSpec((D, D), lambda i: (0, 0)),
          pl.BlockSpec((1, D), lambda i: (0, 0)),
      ],
      out_specs=pl.BlockSpec((RB, D), lambda i: (i, 0)),
      out_shape=jax.ShapeDtypeStruct((N, D), jnp.float32),
      interpret=_INTERPRET,
  )(s1, y1, dinv, w2, b1)


def _k6_body(s_ref, y2_ref, dinv_ref, b_ref, out_ref):
  out_ref[...] = (dinv_ref[...] * (s_ref[0] + s_ref[1] + y2_ref[...])
                  + b_ref[...])


def _k6(s2, y2, dinv, b2):
  return pl.pallas_call(
      _k6_body,
      grid=(N // RB,),
      in_specs=[
          pl.BlockSpec((NC, RB, D), lambda i: (0, i, 0)),
          pl.BlockSpec((RB, D), lambda i: (i, 0)),
          pl.BlockSpec((RB, 1), lambda i: (i, 0)),
          pl.BlockSpec((1, D), lambda i: (0, 0)),
      ],
      out_specs=pl.BlockSpec((RB, D), lambda i: (i, 0)),
      out_shape=jax.ShapeDtypeStruct((N, D), jnp.float32),
      interpret=_INTERPRET,
  )(s2, y2, dinv, b2)


def kernel(node_feature, edge_index, W1, b1, W2, b2):
  src = edge_index[0]
  dst = edge_index[1]
  zeros2d = jnp.zeros((WIN, D), jnp.float32)

  degp = _deg_call()(dst)
  dinv, y1 = _k2(degp.reshape(NC, N, 1), node_feature, W1)
  s1 = _agg_call()(y1, src, dst, zeros2d)
  y2 = _k4(s1, y1, dinv, W2, b1.reshape(1, D))
  s2 = _agg_call()(y2, src, dst, zeros2d)
  return _k6(s2, y2, dinv, b2.reshape(1, D))


# speedup vs baseline: 1.1321x; 1.1050x over previous
import functools

import jax
import jax.numpy as jnp
from jax import lax
from jax.experimental import pallas as pl
from jax.experimental.pallas import tpu as pltpu
from jax.experimental.pallas import tpu_sc as plsc

N = 10000
E = 320000
D = 128

NC = 2
NS = 16
NW = NC * NS
WIN = 128
WT = E // WIN
WPT = WT // NW
EXTRA = WT - WPT * NW
NBUF = 3

DCH = 632
DCH_LAST = N - (NS - 1) * DCH
TAIL = DCH - 4 * WIN
TAIL_LAST = DCH_LAST - 4 * WIN

_INTERPRET = False


def _mesh():
  return plsc.VectorSubcoreMesh(
      core_axis_name="c", subcore_axis_name="s",
      num_cores=NC, num_subcores=NS)


def _worker(cid, tid):
  wid = cid * NS + tid
  ebase = (wid * WPT + jnp.minimum(wid, EXTRA)) * WIN
  has_extra = wid < EXTRA
  return ebase, has_extra


def _deg_kernel(dst_hbm, out_hbm, didx, ones_v, zbuf, acc, isem, ssem):
  cid = lax.axis_index("c")
  tid = lax.axis_index("s")
  ebase, has_extra = _worker(cid, tid)

  def load(w, carry):
    pltpu.async_copy(dst_hbm.at[pl.ds(ebase + w * WIN, WIN)],
                     didx.at[w], isem)
    return carry
  lax.fori_loop(0, WPT, load, 0)

  @pl.when(has_extra)
  def _():
    pltpu.async_copy(dst_hbm.at[pl.ds(ebase + WPT * WIN, WIN)],
                     didx.at[WPT], isem)

  for j in range(WIN // 16):
    ones_v[pl.ds(j * 16, 16)] = jnp.full((16,), 1.0, jnp.float32)
    zbuf[pl.ds(j * 16, 16)] = jnp.zeros((16,), jnp.float32)

  cbase = tid * DCH
  for j in range(4):
    pltpu.sync_copy(zbuf, acc.at[pl.ds(cbase + j * WIN, WIN)])

  @pl.when(tid < NS - 1)
  def _():
    pltpu.sync_copy(zbuf.at[pl.ds(0, TAIL)],
                    acc.at[pl.ds(cbase + 4 * WIN, TAIL)])

  @pl.when(tid == NS - 1)
  def _():
    pltpu.sync_copy(zbuf.at[pl.ds(0, TAIL_LAST)],
                    acc.at[pl.ds(cbase + 4 * WIN, TAIL_LAST)])

  def idrain(w, carry):
    pltpu.make_async_copy(dst_hbm.at[pl.ds(0, WIN)], didx.at[0],
                          isem).wait()
    return carry
  lax.fori_loop(0, WPT, idrain, 0)

  @pl.when(has_extra)
  def _():
    pltpu.make_async_copy(dst_hbm.at[pl.ds(0, WIN)], didx.at[0],
                          isem).wait()

  plsc.subcore_barrier()

  def fire(w, carry):
    pltpu.async_copy(ones_v, acc.at[didx.at[w]], ssem, add=True)
    return carry
  lax.fori_loop(0, WPT, fire, 0)

  @pl.when(has_extra)
  def _():
    pltpu.async_copy(ones_v, acc.at[didx.at[WPT]], ssem, add=True)

  def sdrain(w, carry):
    pltpu.make_async_copy(dst_hbm.at[pl.ds(0, WIN)], didx.at[0],
                          ssem).wait()
    return carry
  lax.fori_loop(0, WPT, sdrain, 0)

  @pl.when(has_extra)
  def _():
    pltpu.make_async_copy(dst_hbm.at[pl.ds(0, WIN)], didx.at[0],
                          ssem).wait()

  plsc.subcore_barrier()

  obase = cid * N + tid * DCH
  for j in range(4):
    pltpu.sync_copy(acc.at[pl.ds(cbase + j * WIN, WIN)], zbuf)
    pltpu.sync_copy(zbuf, out_hbm.at[pl.ds(obase + j * WIN, WIN)])

  @pl.when(tid < NS - 1)
  def _():
    pltpu.sync_copy(acc.at[pl.ds(cbase + 4 * WIN, TAIL)],
                    zbuf.at[pl.ds(0, TAIL)])
    pltpu.sync_copy(zbuf.at[pl.ds(0, TAIL)],
                    out_hbm.at[pl.ds(obase + 4 * WIN, TAIL)])

  @pl.when(tid == NS - 1)
  def _():
    pltpu.sync_copy(acc.at[pl.ds(cbase + 4 * WIN, TAIL_LAST)],
                    zbuf.at[pl.ds(0, TAIL_LAST)])
    pltpu.sync_copy(zbuf.at[pl.ds(0, TAIL_LAST)],
                    out_hbm.at[pl.ds(obase + 4 * WIN, TAIL_LAST)])


@functools.cache
def _deg_call():
  return pl.kernel(
      _deg_kernel,
      out_type=jax.ShapeDtypeStruct((NC * N,), jnp.float32),
      mesh=_mesh(),
      scratch_types=[
          pltpu.VMEM((WPT + 1, WIN), jnp.int32),
          pltpu.VMEM((WIN,), jnp.float32),
          pltpu.VMEM((WIN,), jnp.float32),
          pltpu.VMEM_SHARED((N,), jnp.float32),
          pltpu.SemaphoreType.DMA,
          pltpu.SemaphoreType.DMA,
      ],
      interpret=_INTERPRET,
  )


def _agg_kernel(y_hbm, src_hbm, dst_hbm, zeros_hbm, out_hbm,
                sidx, didx, rows0, rows1, rows2, acc,
                isem0, isem1, isem2, gsem0, gsem1, gsem2,
                ssem0, ssem1, ssem2):
  cid = lax.axis_index("c")
  tid = lax.axis_index("s")
  ebase, has_extra = _worker(cid, tid)
  nwt = WPT + has_extra.astype(jnp.int32)

  rows = (rows0, rows1, rows2)
  isem = (isem0, isem1, isem2)
  gsem = (gsem0, gsem1, gsem2)
  ssem = (ssem0, ssem1, ssem2)

  def start_idx(b, w):
    pltpu.async_copy(src_hbm.at[pl.ds(ebase + w * WIN, WIN)],
                     sidx.at[b], isem[b])
    pltpu.async_copy(dst_hbm.at[pl.ds(ebase + w * WIN, WIN)],
                     didx.at[b], isem[b])

  def wait_idx(b):
    pltpu.make_async_copy(src_hbm.at[pl.ds(0, WIN)], sidx.at[0],
                          isem[b]).wait()
    pltpu.make_async_copy(src_hbm.at[pl.ds(0, WIN)], sidx.at[0],
                          isem[b]).wait()

  HW = WIN // 2

  def start_gather(b):
    pltpu.async_copy(y_hbm.at[sidx.at[b].at[pl.ds(0, HW)]],
                     rows[b].at[pl.ds(0, HW)], gsem[b])
    pltpu.async_copy(y_hbm.at[sidx.at[b].at[pl.ds(HW, HW)]],
                     rows[b].at[pl.ds(HW, HW)], gsem[b])

  def wait_gather(b):
    pltpu.make_async_copy(y_hbm.at[pl.ds(0, WIN)], rows[b],
                          gsem[b]).wait()

  def start_scatter(b):
    pltpu.async_copy(rows[b], acc.at[didx.at[b]], ssem[b], add=True)

  def wait_scatter(b):
    pltpu.make_async_copy(y_hbm.at[pl.ds(0, WIN)], rows[b],
                          ssem[b]).wait()

  start_idx(0, 0)
  start_idx(1, 1)

  cbase = tid * DCH
  pltpu.sync_copy(zeros_hbm, rows0)
  for j in range(4):
    pltpu.async_copy(rows0, acc.at[pl.ds(cbase + j * WIN, WIN)], ssem0)

  @pl.when(tid < NS - 1)
  def _():
    pltpu.async_copy(rows0.at[pl.ds(0, TAIL)],
                     acc.at[pl.ds(cbase + 4 * WIN, TAIL)], ssem0)

  @pl.when(tid == NS - 1)
  def _():
    pltpu.async_copy(rows0.at[pl.ds(0, TAIL_LAST)],
                     acc.at[pl.ds(cbase + 4 * WIN, TAIL_LAST)], ssem0)

  for j in range(4):
    pltpu.make_async_copy(y_hbm.at[pl.ds(0, WIN)], rows0, ssem0).wait()

  @pl.when(tid < NS - 1)
  def _():
    pltpu.make_async_copy(y_hbm.at[pl.ds(0, TAIL)],
                          rows0.at[pl.ds(0, TAIL)], ssem0).wait()

  @pl.when(tid == NS - 1)
  def _():
    pltpu.make_async_copy(y_hbm.at[pl.ds(0, TAIL_LAST)],
                          rows0.at[pl.ds(0, TAIL_LAST)], ssem0).wait()

  plsc.subcore_barrier()

  wait_idx(0)
  start_gather(0)

  def group(g, carry):
    for b in range(NBUF):
      w = NBUF * g + b
      b1 = (b + 1) % NBUF
      b2 = (b + 2) % NBUF
      w1 = w + 1
      w2 = w + 2

      @pl.when(w2 < nwt)
      def _():
        @pl.when(w2 >= NBUF)
        def _():
          wait_scatter(b2)
        start_idx(b2, w2)

      @pl.when(w1 < nwt)
      def _():
        wait_idx(b1)
        start_gather(b1)

      wait_gather(b)
      start_scatter(b)
    return carry

  lax.fori_loop(0, WPT // NBUF, group, 0)

  @pl.when(has_extra)
  def _():
    wait_gather(0)
    start_scatter(0)

  for b in range(NBUF):
    wait_scatter(b)

  plsc.subcore_barrier()

  def wload(b, k):
    pltpu.async_copy(acc.at[pl.ds(cbase + k * WIN, WIN)], rows[b],
                     gsem[b])

  def wload_wait(b):
    pltpu.make_async_copy(y_hbm.at[pl.ds(0, WIN)], rows[b],
                          gsem[b]).wait()

  def wstore(b, k):
    pltpu.async_copy(rows[b], out_hbm.at[cid, pl.ds(cbase + k * WIN, WIN)],
                     ssem[b])

  def wstore_wait(b):
    pltpu.make_async_copy(y_hbm.at[pl.ds(0, WIN)], rows[b],
                          ssem[b]).wait()

  for b in range(3):
    wload(b, b)
  for b in range(3):
    wload_wait(b)
    wstore(b, b)
  wstore_wait(0)
  wload(0, 3)
  wload_wait(0)
  wstore(0, 3)
  wstore_wait(1)

  @pl.when(tid < NS - 1)
  def _():
    pltpu.async_copy(acc.at[pl.ds(cbase + 4 * WIN, TAIL)],
                     rows1.at[pl.ds(0, TAIL)], gsem1)
    pltpu.make_async_copy(y_hbm.at[pl.ds(0, TAIL)],
                          rows1.at[pl.ds(0, TAIL)], gsem1).wait()
    pltpu.async_copy(rows1.at[pl.ds(0, TAIL)],
                     out_hbm.at[cid, pl.ds(cbase + 4 * WIN, TAIL)], ssem1)
    pltpu.make_async_copy(y_hbm.at[pl.ds(0, TAIL)],
                          rows1.at[pl.ds(0, TAIL)], ssem1).wait()

  @pl.when(tid == NS - 1)
  def _():
    pltpu.async_copy(acc.at[pl.ds(cbase + 4 * WIN, TAIL_LAST)],
                     rows1.at[pl.ds(0, TAIL_LAST)], gsem1)
    pltpu.make_async_copy(y_hbm.at[pl.ds(0, TAIL_LAST)],
                          rows1.at[pl.ds(0, TAIL_LAST)], gsem1).wait()
    pltpu.async_copy(rows1.at[pl.ds(0, TAIL_LAST)],
                     out_hbm.at[cid, pl.ds(cbase + 4 * WIN, TAIL_LAST)],
                     ssem1)
    pltpu.make_async_copy(y_hbm.at[pl.ds(0, TAIL_LAST)],
                          rows1.at[pl.ds(0, TAIL_LAST)], ssem1).wait()

  wstore_wait(2)
  wstore_wait(0)


@functools.cache
def _agg_call():
  return pl.kernel(
      _agg_kernel,
      out_type=jax.ShapeDtypeStruct((NC, N, D), jnp.float32),
      mesh=_mesh(),
      scratch_types=[
          pltpu.VMEM((NBUF, WIN), jnp.int32),
          pltpu.VMEM((NBUF, WIN), jnp.int32),
          pltpu.VMEM((WIN, D), jnp.float32),
          pltpu.VMEM((WIN, D), jnp.float32),
          pltpu.VMEM((WIN, D), jnp.float32),
          pltpu.VMEM_SHARED((N, D), jnp.float32),
          pltpu.SemaphoreType.DMA,
          pltpu.SemaphoreType.DMA,
          pltpu.SemaphoreType.DMA,
          pltpu.SemaphoreType.DMA,
          pltpu.SemaphoreType.DMA,
          pltpu.SemaphoreType.DMA,
          pltpu.SemaphoreType.DMA,
          pltpu.SemaphoreType.DMA,
          pltpu.SemaphoreType.DMA,
      ],
      interpret=_INTERPRET,
  )


RB = 2000


def _dot(a, b):
  return lax.dot_general(a, b, (((1,), (0,)), ((), ())),
                         preferred_element_type=jnp.float32)


def _k2_body(degp_ref, x_ref, w_ref, dinv_ref, y1_ref):
  d = degp_ref[0] + degp_ref[1] + 1.0
  dinv = lax.rsqrt(jnp.maximum(d, 1.0))
  dinv_ref[...] = dinv
  y1_ref[...] = _dot(x_ref[...], w_ref[...]) * dinv


def _k2(degp, x, w1):
  return pl.pallas_call(
      _k2_body,
      grid=(N // RB,),
      in_specs=[
          pl.BlockSpec((NC, RB, 1), lambda i: (0, i, 0)),
          pl.BlockSpec((RB, D), lambda i: (i, 0)),
          pl.BlockSpec((D, D), lambda i: (0, 0)),
      ],
      out_specs=[
          pl.BlockSpec((RB, 1), lambda i: (i, 0)),
          pl.BlockSpec((RB, D), lambda i: (i, 0)),
      ],
      out_shape=[
          jax.ShapeDtypeStruct((N, 1), jnp.float32),
          jax.ShapeDtypeStruct((N, D), jnp.float32),
      ],
      interpret=_INTERPRET,
  )(degp, x, w1)


def _k4_body(s_ref, y1_ref, dinv_ref, w_ref, b_ref, y2_ref):
  dinv = dinv_ref[...]
  h1 = dinv * (s_ref[0] + s_ref[1] + y1_ref[...]) + b_ref[...]
  y2_ref[...] = _dot(h1, w_ref[...]) * dinv


def _k4(s1, y1, dinv, w2, b1):
  return pl.pallas_call(
      _k4_body,
      grid=(N // RB,),
      in_specs=[
          pl.BlockSpec((NC, RB, D), lambda i: (0, i, 0)),
          pl.BlockSpec((RB, D), lambda i: (i, 0)),
          pl.BlockSpec((RB, 1), lambda i: (i, 0)),
          pl.BlockSpec((D, D), lambda i: (0, 0)),
          pl.BlockSpec((1, D), lambda i: (0, 0)),
      ],
      out_specs=pl.BlockSpec((RB, D), lambda i: (i, 0)),
      out_shape=jax.ShapeDtypeStruct((N, D), jnp.float32),
      interpret=_INTERPRET,
  )(s1, y1, dinv, w2, b1)


def _k6_body(s_ref, y2_ref, dinv_ref, b_ref, out_ref):
  out_ref[...] = (dinv_ref[...] * (s_ref[0] + s_ref[1] + y2_ref[...])
                  + b_ref[...])


def _k6(s2, y2, dinv, b2):
  return pl.pallas_call(
      _k6_body,
      grid=(N // RB,),
      in_specs=[
          pl.BlockSpec((NC, RB, D), lambda i: (0, i, 0)),
          pl.BlockSpec((RB, D), lambda i: (i, 0)),
          pl.BlockSpec((RB, 1), lambda i: (i, 0)),
          pl.BlockSpec((1, D), lambda i: (0, 0)),
      ],
      out_specs=pl.BlockSpec((RB, D), lambda i: (i, 0)),
      out_shape=jax.ShapeDtypeStruct((N, D), jnp.float32),
      interpret=_INTERPRET,
  )(s2, y2, dinv, b2)


def kernel(node_feature, edge_index, W1, b1, W2, b2):
  src = edge_index[0]
  dst = edge_index[1]
  zeros2d = jnp.zeros((WIN, D), jnp.float32)

  degp = _deg_call()(dst)
  dinv, y1 = _k2(degp.reshape(NC, N, 1), node_feature, W1)
  s1 = _agg_call()(y1, src, dst, zeros2d)
  y2 = _k4(s1, y1, dinv, W2, b1.reshape(1, D))
  s2 = _agg_call()(y2, src, dst, zeros2d)
  return _k6(s2, y2, dinv, b2.reshape(1, D))


# speedup vs baseline: 1.1908x; 1.0519x over previous
import functools

import jax
import jax.numpy as jnp
from jax import lax
from jax.experimental import pallas as pl
from jax.experimental.pallas import tpu as pltpu
from jax.experimental.pallas import tpu_sc as plsc

N = 10000
E = 320000
D = 128

NC = 2
NS = 16
NW = NC * NS
WIN = 128
WT = E // WIN
WPT = WT // NW
EXTRA = WT - WPT * NW
NBUF = 3

DCH = 632
DCH_LAST = N - (NS - 1) * DCH
TAIL = DCH - 4 * WIN
TAIL_LAST = DCH_LAST - 4 * WIN

_INTERPRET = False


def _mesh():
  return plsc.VectorSubcoreMesh(
      core_axis_name="c", subcore_axis_name="s",
      num_cores=NC, num_subcores=NS)


def _worker(cid, tid):
  wid = cid * NS + tid
  ebase = (wid * WPT + jnp.minimum(wid, EXTRA)) * WIN
  has_extra = wid < EXTRA
  return ebase, has_extra


def _deg_kernel(ei_hbm, out_hbm, didx, ones_v, zbuf, acc, isem, ssem):
  cid = lax.axis_index("c")
  tid = lax.axis_index("s")
  ebase, has_extra = _worker(cid, tid)

  def load(w, carry):
    pltpu.async_copy(ei_hbm.at[:, pl.ds(ebase + w * WIN, WIN)],
                     didx.at[w], isem)
    return carry
  lax.fori_loop(0, WPT, load, 0)

  @pl.when(has_extra)
  def _():
    pltpu.async_copy(ei_hbm.at[:, pl.ds(ebase + WPT * WIN, WIN)],
                     didx.at[WPT], isem)

  for j in range(WIN // 16):
    ones_v[pl.ds(j * 16, 16)] = jnp.full((16,), 1.0, jnp.float32)
    zbuf[pl.ds(j * 16, 16)] = jnp.zeros((16,), jnp.float32)

  cbase = tid * DCH
  for j in range(4):
    pltpu.sync_copy(zbuf, acc.at[pl.ds(cbase + j * WIN, WIN)])

  @pl.when(tid < NS - 1)
  def _():
    pltpu.sync_copy(zbuf.at[pl.ds(0, TAIL)],
                    acc.at[pl.ds(cbase + 4 * WIN, TAIL)])

  @pl.when(tid == NS - 1)
  def _():
    pltpu.sync_copy(zbuf.at[pl.ds(0, TAIL_LAST)],
                    acc.at[pl.ds(cbase + 4 * WIN, TAIL_LAST)])

  def idrain(w, carry):
    pltpu.make_async_copy(ei_hbm.at[:, pl.ds(0, WIN)], didx.at[0],
                          isem).wait()
    return carry
  lax.fori_loop(0, WPT, idrain, 0)

  @pl.when(has_extra)
  def _():
    pltpu.make_async_copy(ei_hbm.at[:, pl.ds(0, WIN)], didx.at[0],
                          isem).wait()

  plsc.subcore_barrier()

  def fire(w, carry):
    pltpu.async_copy(ones_v, acc.at[didx.at[w, 1]], ssem, add=True)
    return carry
  lax.fori_loop(0, WPT, fire, 0)

  @pl.when(has_extra)
  def _():
    pltpu.async_copy(ones_v, acc.at[didx.at[WPT, 1]], ssem, add=True)

  def sdrain(w, carry):
    pltpu.make_async_copy(ei_hbm.at[0, pl.ds(0, WIN)], didx.at[0, 0],
                          ssem).wait()
    return carry
  lax.fori_loop(0, WPT, sdrain, 0)

  @pl.when(has_extra)
  def _():
    pltpu.make_async_copy(ei_hbm.at[0, pl.ds(0, WIN)], didx.at[0, 0],
                          ssem).wait()

  plsc.subcore_barrier()

  obase = cid * N + tid * DCH
  for j in range(4):
    pltpu.sync_copy(acc.at[pl.ds(cbase + j * WIN, WIN)], zbuf)
    pltpu.sync_copy(zbuf, out_hbm.at[pl.ds(obase + j * WIN, WIN)])

  @pl.when(tid < NS - 1)
  def _():
    pltpu.sync_copy(acc.at[pl.ds(cbase + 4 * WIN, TAIL)],
                    zbuf.at[pl.ds(0, TAIL)])
    pltpu.sync_copy(zbuf.at[pl.ds(0, TAIL)],
                    out_hbm.at[pl.ds(obase + 4 * WIN, TAIL)])

  @pl.when(tid == NS - 1)
  def _():
    pltpu.sync_copy(acc.at[pl.ds(cbase + 4 * WIN, TAIL_LAST)],
                    zbuf.at[pl.ds(0, TAIL_LAST)])
    pltpu.sync_copy(zbuf.at[pl.ds(0, TAIL_LAST)],
                    out_hbm.at[pl.ds(obase + 4 * WIN, TAIL_LAST)])


@functools.cache
def _deg_call():
  return pl.kernel(
      _deg_kernel,
      out_type=jax.ShapeDtypeStruct((NC * N,), jnp.float32),
      mesh=_mesh(),
      scratch_types=[
          pltpu.VMEM((WPT + 1, 2, WIN), jnp.int32),
          pltpu.VMEM((WIN,), jnp.float32),
          pltpu.VMEM((WIN,), jnp.float32),
          pltpu.VMEM_SHARED((N,), jnp.float32),
          pltpu.SemaphoreType.DMA,
          pltpu.SemaphoreType.DMA,
      ],
      interpret=_INTERPRET,
  )


def _agg_kernel(y_hbm, ei_hbm, zeros_hbm, out_hbm,
                eidx, rows0, rows1, rows2, acc,
                isem0, isem1, isem2, gsem0, gsem1, gsem2,
                ssem0, ssem1, ssem2):
  cid = lax.axis_index("c")
  tid = lax.axis_index("s")
  ebase, has_extra = _worker(cid, tid)
  nwt = WPT + has_extra.astype(jnp.int32)

  rows = (rows0, rows1, rows2)
  isem = (isem0, isem1, isem2)
  gsem = (gsem0, gsem1, gsem2)
  ssem = (ssem0, ssem1, ssem2)

  def start_idx(b, w):
    pltpu.async_copy(ei_hbm.at[:, pl.ds(ebase + w * WIN, WIN)],
                     eidx.at[b], isem[b])

  def wait_idx(b):
    pltpu.make_async_copy(ei_hbm.at[:, pl.ds(0, WIN)], eidx.at[0],
                          isem[b]).wait()

  HW = WIN // 2

  def start_gather(b):
    pltpu.async_copy(y_hbm.at[eidx.at[b, 0].at[pl.ds(0, HW)]],
                     rows[b].at[pl.ds(0, HW)], gsem[b])
    pltpu.async_copy(y_hbm.at[eidx.at[b, 0].at[pl.ds(HW, HW)]],
                     rows[b].at[pl.ds(HW, HW)], gsem[b])

  def wait_gather(b):
    pltpu.make_async_copy(y_hbm.at[pl.ds(0, WIN)], rows[b],
                          gsem[b]).wait()

  def start_scatter(b):
    pltpu.async_copy(rows[b], acc.at[eidx.at[b, 1]], ssem[b], add=True)

  def wait_scatter(b):
    pltpu.make_async_copy(y_hbm.at[pl.ds(0, WIN)], rows[b],
                          ssem[b]).wait()

  start_idx(0, 0)
  start_idx(1, 1)

  cbase = tid * DCH
  pltpu.sync_copy(zeros_hbm, rows0)
  for j in range(4):
    pltpu.async_copy(rows0, acc.at[pl.ds(cbase + j * WIN, WIN)], ssem0)

  @pl.when(tid < NS - 1)
  def _():
    pltpu.async_copy(rows0.at[pl.ds(0, TAIL)],
                     acc.at[pl.ds(cbase + 4 * WIN, TAIL)], ssem0)

  @pl.when(tid == NS - 1)
  def _():
    pltpu.async_copy(rows0.at[pl.ds(0, TAIL_LAST)],
                     acc.at[pl.ds(cbase + 4 * WIN, TAIL_LAST)], ssem0)

  for j in range(4):
    pltpu.make_async_copy(y_hbm.at[pl.ds(0, WIN)], rows0, ssem0).wait()

  @pl.when(tid < NS - 1)
  def _():
    pltpu.make_async_copy(y_hbm.at[pl.ds(0, TAIL)],
                          rows0.at[pl.ds(0, TAIL)], ssem0).wait()

  @pl.when(tid == NS - 1)
  def _():
    pltpu.make_async_copy(y_hbm.at[pl.ds(0, TAIL_LAST)],
                          rows0.at[pl.ds(0, TAIL_LAST)], ssem0).wait()

  plsc.subcore_barrier()

  wait_idx(0)
  start_gather(0)

  def group(g, carry):
    for b in range(NBUF):
      w = NBUF * g + b
      b1 = (b + 1) % NBUF
      b2 = (b + 2) % NBUF
      w1 = w + 1
      w2 = w + 2

      @pl.when(w2 < nwt)
      def _():
        @pl.when(w2 >= NBUF)
        def _():
          wait_scatter(b2)
        start_idx(b2, w2)

      @pl.when(w1 < nwt)
      def _():
        wait_idx(b1)
        start_gather(b1)

      wait_gather(b)
      start_scatter(b)
    return carry

  lax.fori_loop(0, WPT // NBUF, group, 0)

  @pl.when(has_extra)
  def _():
    wait_gather(0)
    start_scatter(0)

  for b in range(NBUF):
    wait_scatter(b)

  plsc.subcore_barrier()

  def wload(b, k):
    pltpu.async_copy(acc.at[pl.ds(cbase + k * WIN, WIN)], rows[b],
                     gsem[b])

  def wload_wait(b):
    pltpu.make_async_copy(y_hbm.at[pl.ds(0, WIN)], rows[b],
                          gsem[b]).wait()

  def wstore(b, k):
    pltpu.async_copy(rows[b], out_hbm.at[cid, pl.ds(cbase + k * WIN, WIN)],
                     ssem[b])

  def wstore_wait(b):
    pltpu.make_async_copy(y_hbm.at[pl.ds(0, WIN)], rows[b],
                          ssem[b]).wait()

  for b in range(3):
    wload(b, b)
  for b in range(3):
    wload_wait(b)
    wstore(b, b)
  wstore_wait(0)
  wload(0, 3)
  wload_wait(0)
  wstore(0, 3)
  wstore_wait(1)

  @pl.when(tid < NS - 1)
  def _():
    pltpu.async_copy(acc.at[pl.ds(cbase + 4 * WIN, TAIL)],
                     rows1.at[pl.ds(0, TAIL)], gsem1)
    pltpu.make_async_copy(y_hbm.at[pl.ds(0, TAIL)],
                          rows1.at[pl.ds(0, TAIL)], gsem1).wait()
    pltpu.async_copy(rows1.at[pl.ds(0, TAIL)],
                     out_hbm.at[cid, pl.ds(cbase + 4 * WIN, TAIL)], ssem1)
    pltpu.make_async_copy(y_hbm.at[pl.ds(0, TAIL)],
                          rows1.at[pl.ds(0, TAIL)], ssem1).wait()

  @pl.when(tid == NS - 1)
  def _():
    pltpu.async_copy(acc.at[pl.ds(cbase + 4 * WIN, TAIL_LAST)],
                     rows1.at[pl.ds(0, TAIL_LAST)], gsem1)
    pltpu.make_async_copy(y_hbm.at[pl.ds(0, TAIL_LAST)],
                          rows1.at[pl.ds(0, TAIL_LAST)], gsem1).wait()
    pltpu.async_copy(rows1.at[pl.ds(0, TAIL_LAST)],
                     out_hbm.at[cid, pl.ds(cbase + 4 * WIN, TAIL_LAST)],
                     ssem1)
    pltpu.make_async_copy(y_hbm.at[pl.ds(0, TAIL_LAST)],
                          rows1.at[pl.ds(0, TAIL_LAST)], ssem1).wait()

  wstore_wait(2)
  wstore_wait(0)


@functools.cache
def _agg_call():
  return pl.kernel(
      _agg_kernel,
      out_type=jax.ShapeDtypeStruct((NC, N, D), jnp.float32),
      mesh=_mesh(),
      scratch_types=[
          pltpu.VMEM((NBUF, 2, WIN), jnp.int32),
          pltpu.VMEM((WIN, D), jnp.float32),
          pltpu.VMEM((WIN, D), jnp.float32),
          pltpu.VMEM((WIN, D), jnp.float32),
          pltpu.VMEM_SHARED((N, D), jnp.float32),
          pltpu.SemaphoreType.DMA,
          pltpu.SemaphoreType.DMA,
          pltpu.SemaphoreType.DMA,
          pltpu.SemaphoreType.DMA,
          pltpu.SemaphoreType.DMA,
          pltpu.SemaphoreType.DMA,
          pltpu.SemaphoreType.DMA,
          pltpu.SemaphoreType.DMA,
          pltpu.SemaphoreType.DMA,
      ],
      interpret=_INTERPRET,
  )


RB = 2000


def _dot(a, b):
  return lax.dot_general(a, b, (((1,), (0,)), ((), ())),
                         preferred_element_type=jnp.float32)


def _k2_body(degp_ref, x_ref, w_ref, dinv_ref, y1_ref):
  d = degp_ref[0] + degp_ref[1] + 1.0
  dinv = lax.rsqrt(jnp.maximum(d, 1.0))
  dinv_ref[...] = dinv
  y1_ref[...] = _dot(x_ref[...], w_ref[...]) * dinv


def _k2(degp, x, w1):
  return pl.pallas_call(
      _k2_body,
      grid=(N // RB,),
      in_specs=[
          pl.BlockSpec((NC, RB, 1), lambda i: (0, i, 0)),
          pl.BlockSpec((RB, D), lambda i: (i, 0)),
          pl.BlockSpec((D, D), lambda i: (0, 0)),
      ],
      out_specs=[
          pl.BlockSpec((RB, 1), lambda i: (i, 0)),
          pl.BlockSpec((RB, D), lambda i: (i, 0)),
      ],
      out_shape=[
          jax.ShapeDtypeStruct((N, 1), jnp.float32),
          jax.ShapeDtypeStruct((N, D), jnp.float32),
      ],
      interpret=_INTERPRET,
  )(degp, x, w1)


def _k4_body(s_ref, y1_ref, dinv_ref, w_ref, b_ref, y2_ref):
  dinv = dinv_ref[...]
  h1 = dinv * (s_ref[0] + s_ref[1] + y1_ref[...]) + b_ref[...]
  y2_ref[...] = _dot(h1, w_ref[...]) * dinv


def _k4(s1, y1, dinv, w2, b1):
  return pl.pallas_call(
      _k4_body,
      grid=(N // RB,),
      in_specs=[
          pl.BlockSpec((NC, RB, D), lambda i: (0, i, 0)),
          pl.BlockSpec((RB, D), lambda i: (i, 0)),
          pl.BlockSpec((RB, 1), lambda i: (i, 0)),
          pl.BlockSpec((D, D), lambda i: (0, 0)),
          pl.BlockSpec((1, D), lambda i: (0, 0)),
      ],
      out_specs=pl.BlockSpec((RB, D), lambda i: (i, 0)),
      out_shape=jax.ShapeDtypeStruct((N, D), jnp.float32),
      interpret=_INTERPRET,
  )(s1, y1, dinv, w2, b1)


def _k6_body(s_ref, y2_ref, dinv_ref, b_ref, out_ref):
  out_ref[...] = (dinv_ref[...] * (s_ref[0] + s_ref[1] + y2_ref[...])
                  + b_ref[...])


def _k6(s2, y2, dinv, b2):
  return pl.pallas_call(
      _k6_body,
      grid=(N // RB,),
      in_specs=[
          pl.BlockSpec((NC, RB, D), lambda i: (0, i, 0)),
          pl.BlockSpec((RB, D), lambda i: (i, 0)),
          pl.BlockSpec((RB, 1), lambda i: (i, 0)),
          pl.BlockSpec((1, D), lambda i: (0, 0)),
      ],
      out_specs=pl.BlockSpec((RB, D), lambda i: (i, 0)),
      out_shape=jax.ShapeDtypeStruct((N, D), jnp.float32),
      interpret=_INTERPRET,
  )(s2, y2, dinv, b2)


def kernel(node_feature, edge_index, W1, b1, W2, b2):
  zeros2d = jnp.zeros((WIN, D), jnp.float32)

  degp = _deg_call()(edge_index)
  dinv, y1 = _k2(degp.reshape(NC, N, 1), node_feature, W1)
  s1 = _agg_call()(y1, edge_index, zeros2d)
  y2 = _k4(s1, y1, dinv, W2, b1.reshape(1, D))
  s2 = _agg_call()(y2, edge_index, zeros2d)
  return _k6(s2, y2, dinv, b2.reshape(1, D))


# speedup vs baseline: 1.2534x; 1.0526x over previous
import functools

import jax
import jax.numpy as jnp
from jax import lax
from jax.experimental import pallas as pl
from jax.experimental.pallas import tpu as pltpu
from jax.experimental.pallas import tpu_sc as plsc

N = 10000
E = 320000
D = 128

NC = 2
NS = 16
NW = NC * NS
WIN = 128
WT = E // WIN
WPT = WT // NW
EXTRA = WT - WPT * NW
NBUF = 3

DCH = 632
DCH_LAST = N - (NS - 1) * DCH
TAIL = DCH - 4 * WIN
TAIL_LAST = DCH_LAST - 4 * WIN

_INTERPRET = False


def _mesh():
  return plsc.VectorSubcoreMesh(
      core_axis_name="c", subcore_axis_name="s",
      num_cores=NC, num_subcores=NS)


def _worker(cid, tid):
  wid = cid * NS + tid
  ebase = (wid * WPT + jnp.minimum(wid, EXTRA)) * WIN
  has_extra = wid < EXTRA
  return ebase, has_extra


def _deg_kernel(ei_hbm, out_hbm, didx, ones_v, zbuf, acc, isem, ssem):
  cid = lax.axis_index("c")
  tid = lax.axis_index("s")
  ebase, has_extra = _worker(cid, tid)

  def load(w, carry):
    pltpu.async_copy(ei_hbm.at[:, pl.ds(ebase + w * WIN, WIN)],
                     didx.at[w], isem)
    return carry
  lax.fori_loop(0, WPT, load, 0)

  @pl.when(has_extra)
  def _():
    pltpu.async_copy(ei_hbm.at[:, pl.ds(ebase + WPT * WIN, WIN)],
                     didx.at[WPT], isem)

  for j in range(WIN // 16):
    ones_v[pl.ds(j * 16, 16)] = jnp.full((16,), 1.0, jnp.float32)
    zbuf[pl.ds(j * 16, 16)] = jnp.zeros((16,), jnp.float32)

  cbase = tid * DCH
  for j in range(4):
    pltpu.sync_copy(zbuf, acc.at[pl.ds(cbase + j * WIN, WIN)])

  @pl.when(tid < NS - 1)
  def _():
    pltpu.sync_copy(zbuf.at[pl.ds(0, TAIL)],
                    acc.at[pl.ds(cbase + 4 * WIN, TAIL)])

  @pl.when(tid == NS - 1)
  def _():
    pltpu.sync_copy(zbuf.at[pl.ds(0, TAIL_LAST)],
                    acc.at[pl.ds(cbase + 4 * WIN, TAIL_LAST)])

  def idrain(w, carry):
    pltpu.make_async_copy(ei_hbm.at[:, pl.ds(0, WIN)], didx.at[0],
                          isem).wait()
    return carry
  lax.fori_loop(0, WPT, idrain, 0)

  @pl.when(has_extra)
  def _():
    pltpu.make_async_copy(ei_hbm.at[:, pl.ds(0, WIN)], didx.at[0],
                          isem).wait()

  plsc.subcore_barrier()

  def fire(w, carry):
    pltpu.async_copy(ones_v, acc.at[didx.at[w, 1]], ssem, add=True)
    return carry
  lax.fori_loop(0, WPT, fire, 0)

  @pl.when(has_extra)
  def _():
    pltpu.async_copy(ones_v, acc.at[didx.at[WPT, 1]], ssem, add=True)

  def sdrain(w, carry):
    pltpu.make_async_copy(ei_hbm.at[0, pl.ds(0, WIN)], didx.at[0, 0],
                          ssem).wait()
    return carry
  lax.fori_loop(0, WPT, sdrain, 0)

  @pl.when(has_extra)
  def _():
    pltpu.make_async_copy(ei_hbm.at[0, pl.ds(0, WIN)], didx.at[0, 0],
                          ssem).wait()

  plsc.subcore_barrier()

  obase = cid * NP + tid * DCH
  for j in range(4):
    pltpu.sync_copy(acc.at[pl.ds(cbase + j * WIN, WIN)], zbuf)
    pltpu.sync_copy(zbuf, out_hbm.at[pl.ds(obase + j * WIN, WIN)])

  @pl.when(tid < NS - 1)
  def _():
    pltpu.sync_copy(acc.at[pl.ds(cbase + 4 * WIN, TAIL)],
                    zbuf.at[pl.ds(0, TAIL)])
    pltpu.sync_copy(zbuf.at[pl.ds(0, TAIL)],
                    out_hbm.at[pl.ds(obase + 4 * WIN, TAIL)])

  @pl.when(tid == NS - 1)
  def _():
    pltpu.sync_copy(acc.at[pl.ds(cbase + 4 * WIN, TAIL_LAST)],
                    zbuf.at[pl.ds(0, TAIL_LAST)])
    pltpu.sync_copy(zbuf.at[pl.ds(0, TAIL_LAST)],
                    out_hbm.at[pl.ds(obase + 4 * WIN, TAIL_LAST)])


@functools.cache
def _deg_call():
  return pl.kernel(
      _deg_kernel,
      out_type=jax.ShapeDtypeStruct((NC * NP,), jnp.float32),
      mesh=_mesh(),
      scratch_types=[
          pltpu.VMEM((WPT + 1, 2, WIN), jnp.int32),
          pltpu.VMEM((WIN,), jnp.float32),
          pltpu.VMEM((WIN,), jnp.float32),
          pltpu.VMEM_SHARED((N,), jnp.float32),
          pltpu.SemaphoreType.DMA,
          pltpu.SemaphoreType.DMA,
      ],
      interpret=_INTERPRET,
  )


def _agg_kernel(y_hbm, ei_hbm, zeros_hbm, out_hbm,
                eidx, rows0, rows1, rows2, acc,
                isem0, isem1, isem2, gsem0, gsem1, gsem2,
                ssem0, ssem1, ssem2):
  cid = lax.axis_index("c")
  tid = lax.axis_index("s")
  ebase, has_extra = _worker(cid, tid)
  nwt = WPT + has_extra.astype(jnp.int32)

  rows = (rows0, rows1, rows2)
  isem = (isem0, isem1, isem2)
  gsem = (gsem0, gsem1, gsem2)
  ssem = (ssem0, ssem1, ssem2)

  def start_idx(b, w):
    pltpu.async_copy(ei_hbm.at[:, pl.ds(ebase + w * WIN, WIN)],
                     eidx.at[b], isem[b])

  def wait_idx(b):
    pltpu.make_async_copy(ei_hbm.at[:, pl.ds(0, WIN)], eidx.at[0],
                          isem[b]).wait()

  HW = WIN // 2

  def start_gather(b):
    pltpu.async_copy(y_hbm.at[eidx.at[b, 0].at[pl.ds(0, HW)]],
                     rows[b].at[pl.ds(0, HW)], gsem[b])
    pltpu.async_copy(y_hbm.at[eidx.at[b, 0].at[pl.ds(HW, HW)]],
                     rows[b].at[pl.ds(HW, HW)], gsem[b])

  def wait_gather(b):
    pltpu.make_async_copy(y_hbm.at[pl.ds(0, WIN)], rows[b],
                          gsem[b]).wait()

  def start_scatter(b):
    pltpu.async_copy(rows[b], acc.at[eidx.at[b, 1]], ssem[b], add=True)

  def wait_scatter(b):
    pltpu.make_async_copy(y_hbm.at[pl.ds(0, WIN)], rows[b],
                          ssem[b]).wait()

  start_idx(0, 0)
  start_idx(1, 1)

  cbase = tid * DCH
  pltpu.sync_copy(zeros_hbm, rows0)
  for j in range(4):
    pltpu.async_copy(rows0, acc.at[pl.ds(cbase + j * WIN, WIN)], ssem0)

  @pl.when(tid < NS - 1)
  def _():
    pltpu.async_copy(rows0.at[pl.ds(0, TAIL)],
                     acc.at[pl.ds(cbase + 4 * WIN, TAIL)], ssem0)

  @pl.when(tid == NS - 1)
  def _():
    pltpu.async_copy(rows0.at[pl.ds(0, TAIL_LAST)],
                     acc.at[pl.ds(cbase + 4 * WIN, TAIL_LAST)], ssem0)

  for j in range(4):
    pltpu.make_async_copy(y_hbm.at[pl.ds(0, WIN)], rows0, ssem0).wait()

  @pl.when(tid < NS - 1)
  def _():
    pltpu.make_async_copy(y_hbm.at[pl.ds(0, TAIL)],
                          rows0.at[pl.ds(0, TAIL)], ssem0).wait()

  @pl.when(tid == NS - 1)
  def _():
    pltpu.make_async_copy(y_hbm.at[pl.ds(0, TAIL_LAST)],
                          rows0.at[pl.ds(0, TAIL_LAST)], ssem0).wait()

  plsc.subcore_barrier()

  wait_idx(0)
  start_gather(0)

  def group(g, carry):
    for b in range(NBUF):
      w = NBUF * g + b
      b1 = (b + 1) % NBUF
      b2 = (b + 2) % NBUF
      w1 = w + 1
      w2 = w + 2

      @pl.when(w2 < nwt)
      def _():
        @pl.when(w2 >= NBUF)
        def _():
          wait_scatter(b2)
        start_idx(b2, w2)

      @pl.when(w1 < nwt)
      def _():
        wait_idx(b1)
        start_gather(b1)

      wait_gather(b)
      start_scatter(b)
    return carry

  lax.fori_loop(0, WPT // NBUF, group, 0)

  @pl.when(has_extra)
  def _():
    wait_gather(0)
    start_scatter(0)

  for b in range(NBUF):
    wait_scatter(b)

  plsc.subcore_barrier()

  def wload(b, k):
    pltpu.async_copy(acc.at[pl.ds(cbase + k * WIN, WIN)], rows[b],
                     gsem[b])

  def wload_wait(b):
    pltpu.make_async_copy(y_hbm.at[pl.ds(0, WIN)], rows[b],
                          gsem[b]).wait()

  def wstore(b, k):
    pltpu.async_copy(rows[b], out_hbm.at[cid, pl.ds(cbase + k * WIN, WIN)],
                     ssem[b])

  def wstore_wait(b):
    pltpu.make_async_copy(y_hbm.at[pl.ds(0, WIN)], rows[b],
                          ssem[b]).wait()

  for b in range(3):
    wload(b, b)
  for b in range(3):
    wload_wait(b)
    wstore(b, b)
  wstore_wait(0)
  wload(0, 3)
  wload_wait(0)
  wstore(0, 3)
  wstore_wait(1)

  @pl.when(tid < NS - 1)
  def _():
    pltpu.async_copy(acc.at[pl.ds(cbase + 4 * WIN, TAIL)],
                     rows1.at[pl.ds(0, TAIL)], gsem1)
    pltpu.make_async_copy(y_hbm.at[pl.ds(0, TAIL)],
                          rows1.at[pl.ds(0, TAIL)], gsem1).wait()
    pltpu.async_copy(rows1.at[pl.ds(0, TAIL)],
                     out_hbm.at[cid, pl.ds(cbase + 4 * WIN, TAIL)], ssem1)
    pltpu.make_async_copy(y_hbm.at[pl.ds(0, TAIL)],
                          rows1.at[pl.ds(0, TAIL)], ssem1).wait()

  @pl.when(tid == NS - 1)
  def _():
    pltpu.async_copy(acc.at[pl.ds(cbase + 4 * WIN, TAIL_LAST)],
                     rows1.at[pl.ds(0, TAIL_LAST)], gsem1)
    pltpu.make_async_copy(y_hbm.at[pl.ds(0, TAIL_LAST)],
                          rows1.at[pl.ds(0, TAIL_LAST)], gsem1).wait()
    pltpu.async_copy(rows1.at[pl.ds(0, TAIL_LAST)],
                     out_hbm.at[cid, pl.ds(cbase + 4 * WIN, TAIL_LAST)],
                     ssem1)
    pltpu.make_async_copy(y_hbm.at[pl.ds(0, TAIL_LAST)],
                          rows1.at[pl.ds(0, TAIL_LAST)], ssem1).wait()

  wstore_wait(2)
  wstore_wait(0)


@functools.cache
def _agg_call():
  return pl.kernel(
      _agg_kernel,
      out_type=jax.ShapeDtypeStruct((NC, N, D), jnp.float32),
      mesh=_mesh(),
      scratch_types=[
          pltpu.VMEM((NBUF, 2, WIN), jnp.int32),
          pltpu.VMEM((WIN, D), jnp.float32),
          pltpu.VMEM((WIN, D), jnp.float32),
          pltpu.VMEM((WIN, D), jnp.float32),
          pltpu.VMEM_SHARED((N, D), jnp.float32),
          pltpu.SemaphoreType.DMA,
          pltpu.SemaphoreType.DMA,
          pltpu.SemaphoreType.DMA,
          pltpu.SemaphoreType.DMA,
          pltpu.SemaphoreType.DMA,
          pltpu.SemaphoreType.DMA,
          pltpu.SemaphoreType.DMA,
          pltpu.SemaphoreType.DMA,
          pltpu.SemaphoreType.DMA,
      ],
      interpret=_INTERPRET,
  )


RB = 2048
NP = 5 * RB
GRID = (NP // RB,)


def _dot(a, b):
  return lax.dot_general(a, b, (((1,), (0,)), ((), ())),
                         preferred_element_type=jnp.float32)


def _k2_body(degp_ref, x_ref, w_ref, dinv_ref, y1_ref):
  off = pl.multiple_of(pl.program_id(0) * RB, 128)
  d = (degp_ref[0, pl.ds(off, RB)] + degp_ref[1, pl.ds(off, RB)]
       + 1.0)
  dinv_lane = lax.rsqrt(jnp.maximum(d, 1.0))
  dinv = jnp.broadcast_to(dinv_lane.reshape(RB, 1), (RB, D))
  dinv_ref[...] = dinv
  y1_ref[...] = _dot(x_ref[...], w_ref[...]) * dinv


def _k2(degp, x, w1):
  return pl.pallas_call(
      _k2_body,
      grid=GRID,
      in_specs=[
          pl.BlockSpec((NC, NP), lambda i: (0, 0)),
          pl.BlockSpec((RB, D), lambda i: (i, 0)),
          pl.BlockSpec((D, D), lambda i: (0, 0)),
      ],
      out_specs=[
          pl.BlockSpec((RB, D), lambda i: (i, 0)),
          pl.BlockSpec((RB, D), lambda i: (i, 0)),
      ],
      out_shape=[
          jax.ShapeDtypeStruct((N, D), jnp.float32),
          jax.ShapeDtypeStruct((N, D), jnp.float32),
      ],
      interpret=_INTERPRET,
  )(degp, x, w1)


def _k4_body(s_ref, y1_ref, dinv_ref, w_ref, b_ref, y2_ref):
  dinv = dinv_ref[...]
  h1 = dinv * (s_ref[0] + s_ref[1] + y1_ref[...]) + b_ref[...]
  y2_ref[...] = _dot(h1, w_ref[...]) * dinv


def _k4(s1, y1, dinv, w2, b1):
  return pl.pallas_call(
      _k4_body,
      grid=GRID,
      in_specs=[
          pl.BlockSpec((NC, RB, D), lambda i: (0, i, 0)),
          pl.BlockSpec((RB, D), lambda i: (i, 0)),
          pl.BlockSpec((RB, D), lambda i: (i, 0)),
          pl.BlockSpec((D, D), lambda i: (0, 0)),
          pl.BlockSpec((1, D), lambda i: (0, 0)),
      ],
      out_specs=pl.BlockSpec((RB, D), lambda i: (i, 0)),
      out_shape=jax.ShapeDtypeStruct((N, D), jnp.float32),
      interpret=_INTERPRET,
  )(s1, y1, dinv, w2, b1)


def _k6_body(s_ref, y2_ref, dinv_ref, b_ref, out_ref):
  out_ref[...] = (dinv_ref[...] * (s_ref[0] + s_ref[1] + y2_ref[...])
                  + b_ref[...])


def _k6(s2, y2, dinv, b2):
  return pl.pallas_call(
      _k6_body,
      grid=GRID,
      in_specs=[
          pl.BlockSpec((NC, RB, D), lambda i: (0, i, 0)),
          pl.BlockSpec((RB, D), lambda i: (i, 0)),
          pl.BlockSpec((RB, D), lambda i: (i, 0)),
          pl.BlockSpec((1, D), lambda i: (0, 0)),
      ],
      out_specs=pl.BlockSpec((RB, D), lambda i: (i, 0)),
      out_shape=jax.ShapeDtypeStruct((N, D), jnp.float32),
      interpret=_INTERPRET,
  )(s2, y2, dinv, b2)


def kernel(node_feature, edge_index, W1, b1, W2, b2):
  zeros2d = jnp.zeros((WIN, D), jnp.float32)

  degp = _deg_call()(edge_index)
  dinv, y1 = _k2(degp.reshape(NC, NP), node_feature, W1)
  s1 = _agg_call()(y1, edge_index, zeros2d)
  y2 = _k4(s1, y1, dinv, W2, b1.reshape(1, D))
  s2 = _agg_call()(y2, edge_index, zeros2d)
  return _k6(s2, y2, dinv, b2.reshape(1, D))
